# baseline (speedup 1.0000x reference)
import functools

import jax
import jax.numpy as jnp
import numpy as np
from jax import lax
from jax.experimental import pallas as pl
from jax.experimental.pallas import tpu as pltpu
from jax.experimental.pallas import tpu_sc as plsc

B, F, D = 16384, 26, 16
V = 2600000
_info = plsc.get_sparse_core_info()
NC, NS, L = _info.num_cores, _info.num_subcores, _info.num_lanes
NW = NC * NS
NFG = 2
FPG = F // NFG
NBS = NW // NFG
BPW = B // NBS

_FIELD_DIMS = [100000] * F
_OFFSETS = np.concatenate([[0], np.cumsum(_FIELD_DIMS)[:-1]]).astype(np.int32)
_OFF_TAB = np.repeat(_OFFSETS[:, None], L, axis=1)


def _sc_kernel(xt_hbm, off_hbm, tt_hbm, out_hbm, idx_v, off_v, idx2_v,
               g0_v, g1_v, gsem, wsem0, wsem1):
    wid = lax.axis_index("s") * NC + lax.axis_index("c")
    fg = wid // NBS
    bs = wid % NBS
    f0 = fg * FPG
    b0 = bs * BPW

    pltpu.sync_copy(xt_hbm.at[pl.ds(f0, FPG), pl.ds(b0, BPW)], idx_v)
    pltpu.sync_copy(off_hbm.at[pl.ds(f0, FPG)], off_v)

    def add_off(fi, carry):
        off_reg = off_v[fi, :]
        def body(j, c):
            sl = pl.ds(j * L, L)
            idx_v[fi, sl] = idx_v[fi, sl] + off_reg
            return c
        return lax.fori_loop(0, BPW // L, body, carry)

    lax.fori_loop(0, FPG, add_off, 0)

    gbufs = (g0_v, g1_v)
    wsems = (wsem0, wsem1)

    def build_idx2(fi):
        def body(j, c):
            sl = pl.ds(j * L, L)
            base = idx_v[fi, sl]
            for d in range(D):
                idx2_v[d, sl] = base + jnp.int32(d * V)
            return c
        lax.fori_loop(0, BPW // L, body, 0)

    def wait_write(buf, wsem):
        pltpu.make_async_copy(buf, out_hbm.at[0, pl.ds(0, D), pl.ds(0, BPW)],
                              wsem).wait()

    for fi in range(FPG):
        p = fi % 2
        buf = gbufs[p]
        if fi >= 2:
            wait_write(buf, wsems[p])
        build_idx2(fi)

        def gfire(d, c):
            pltpu.async_copy(tt_hbm.at[idx2_v.at[d]], buf.at[d], gsem)
            return c
        lax.fori_loop(0, D, gfire, 0)

        def gdrain(d, c):
            pltpu.make_async_copy(tt_hbm.at[pl.ds(0, BPW)], buf.at[d],
                                  gsem).wait()
            return c
        lax.fori_loop(0, D, gdrain, 0)

        pltpu.async_copy(
            buf, out_hbm.at[f0 + fi, pl.ds(0, D), pl.ds(b0, BPW)], wsems[p])

    wait_write(gbufs[(FPG - 1) % 2], wsems[(FPG - 1) % 2])
    wait_write(gbufs[(FPG - 2) % 2], wsems[(FPG - 2) % 2])


@jax.jit
def _run(xt, off, tt_flat):
    return pl.kernel(
        _sc_kernel,
        out_type=jax.ShapeDtypeStruct((F, D, B), jnp.float32),
        mesh=plsc.VectorSubcoreMesh(core_axis_name="c", subcore_axis_name="s"),
        scratch_types=[
            pltpu.VMEM((FPG, BPW), jnp.int32),
            pltpu.VMEM((FPG, L), jnp.int32),
            pltpu.VMEM((D, BPW), jnp.int32),
            pltpu.VMEM((D, BPW), jnp.float32),
            pltpu.VMEM((D, BPW), jnp.float32),
            pltpu.SemaphoreType.DMA,
            pltpu.SemaphoreType.DMA,
            pltpu.SemaphoreType.DMA,
        ],
        compiler_params=pltpu.CompilerParams(use_tc_tiling_on_sc=False),
    )(xt, off, tt_flat)


def kernel(x, table):
    off = jnp.asarray(_OFF_TAB)
    out_fdb = _run(x.T, off, table.T.reshape(-1))
    return out_fdb.transpose(2, 0, 1)


# speedup vs baseline: 3.4753x; 3.4753x over previous
import functools

import jax
import jax.numpy as jnp
import numpy as np
from jax import lax
from jax.experimental import pallas as pl
from jax.experimental.pallas import tpu as pltpu
from jax.experimental.pallas import tpu_sc as plsc

B, F, D = 16384, 26, 16
V = 2600000
_info = plsc.get_sparse_core_info()
NC, NS, L = _info.num_cores, _info.num_subcores, _info.num_lanes
NW = NC * NS
NFG = 2
FPG = F // NFG
NBS = NW // NFG
BPW = B // NBS

_FIELD_DIMS = [100000] * F
_OFFSETS = np.concatenate([[0], np.cumsum(_FIELD_DIMS)[:-1]]).astype(np.int32)
_OFF_TAB = np.repeat(_OFFSETS[:, None], L, axis=1)


def _sc_kernel(*refs):
    xt_hbm, off_hbm = refs[0], refs[1]
    t_rows = refs[2:2 + D]
    out_hbm = refs[2 + D]
    idx_v, off_v, g0_v, g1_v, gsem, wsem0, wsem1 = refs[3 + D:]

    wid = lax.axis_index("s") * NC + lax.axis_index("c")
    fg = wid // NBS
    bs = wid % NBS
    f0 = fg * FPG
    b0 = bs * BPW

    pltpu.sync_copy(xt_hbm.at[pl.ds(f0, FPG), pl.ds(b0, BPW)], idx_v)
    pltpu.sync_copy(off_hbm.at[pl.ds(f0, FPG)], off_v)

    def add_off(fi, carry):
        off_reg = off_v[fi, :]
        def body(j, c):
            sl = pl.ds(j * L, L)
            idx_v[fi, sl] = idx_v[fi, sl] + off_reg
            return c
        return lax.fori_loop(0, BPW // L, body, carry)

    lax.fori_loop(0, FPG, add_off, 0)

    gbufs = (g0_v, g1_v)
    wsems = (wsem0, wsem1)

    def wait_write(buf, wsem):
        pltpu.make_async_copy(buf, out_hbm.at[0, pl.ds(0, D), pl.ds(0, BPW)],
                              wsem).wait()

    for fi in range(FPG):
        p = fi % 2
        buf = gbufs[p]
        if fi >= 2:
            wait_write(buf, wsems[p])
        idx_row = idx_v.at[fi]
        for d in range(D):
            pltpu.async_copy(t_rows[d].at[idx_row], buf.at[d], gsem)
        for d in range(D):
            pltpu.make_async_copy(t_rows[d].at[pl.ds(0, BPW)], buf.at[d],
                                  gsem).wait()
        pltpu.async_copy(
            buf, out_hbm.at[f0 + fi, pl.ds(0, D), pl.ds(b0, BPW)], wsems[p])

    wait_write(gbufs[(FPG - 1) % 2], wsems[(FPG - 1) % 2])
    wait_write(gbufs[(FPG - 2) % 2], wsems[(FPG - 2) % 2])


@jax.jit
def _run(xt, off, *t_rows):
    return pl.kernel(
        _sc_kernel,
        out_type=jax.ShapeDtypeStruct((F, D, B), jnp.float32),
        mesh=plsc.VectorSubcoreMesh(core_axis_name="c", subcore_axis_name="s"),
        scratch_types=[
            pltpu.VMEM((FPG, BPW), jnp.int32),
            pltpu.VMEM((FPG, L), jnp.int32),
            pltpu.VMEM((D, BPW), jnp.float32),
            pltpu.VMEM((D, BPW), jnp.float32),
            pltpu.SemaphoreType.DMA,
            pltpu.SemaphoreType.DMA,
            pltpu.SemaphoreType.DMA,
        ],
        compiler_params=pltpu.CompilerParams(use_tc_tiling_on_sc=False),
    )(xt, off, *t_rows)


def kernel(x, table):
    off = jnp.asarray(_OFF_TAB)
    tt = table.T
    t_rows = [tt[d] for d in range(D)]
    out_fdb = _run(x.T, off, *t_rows)
    return out_fdb.transpose(2, 0, 1)
